# Initial kernel scaffold; baseline (speedup 1.0000x reference)
import jax
import jax.numpy as jnp
from jax.experimental import pallas as pl


def kernel(features_tensor_white, features_tensor_black, is_white_stm_tensor, ft_white_W, ft_white_b, ft_black_W, ft_black_b, out_W, out_b):
    raise NotImplementedError("write your pallas kernel here")


# speedup vs baseline: 2.8540x; 2.8540x over previous
import jax
import jax.numpy as jnp
from jax import lax
from jax.experimental import pallas as pl
from jax.experimental.pallas import tpu as pltpu
from jax.experimental.pallas import tpu_sc as plsc

BATCH = 4096
ACTIVE = 32
HIDDEN = 1024
NCORES = 2
NSUB = 16
NWORK = NCORES * NSUB
BPW = BATCH // NWORK
NCHUNK = HIDDEN // 16
GROWS = 16
NBUF = 2


def _sum_lanes(v):
    lane = lax.iota(jnp.int32, 16)
    dnums = lax.GatherDimensionNumbers(
        offset_dims=(), collapsed_slice_dims=(0,), start_index_map=(0,))
    for m in (8, 4, 2, 1):
        perm = lax.gather(v, (lane ^ m)[:, None], dnums, slice_sizes=(1,),
                          mode=lax.GatherScatterMode.PROMISE_IN_BOUNDS)
        v = v + perm
    return v


def _tree_sum(vals):
    while len(vals) > 1:
        nxt = [vals[j] + vals[j + 1] for j in range(0, len(vals) - 1, 2)]
        if len(vals) % 2:
            nxt.append(vals[-1])
        vals = nxt
    return vals[0]


def _sc_body(fw_hbm, fb_hbm, stm_hbm, ww_hbm, bw_hbm, wb_hbm, bb_hbm, ow_hbm,
             out_hbm,
             idx_v, stm_v, bw_v, bb_v, ow_v, pw1_v, pw2_v, pb1_v, pb2_v,
             out_v, hrow_v, buf, sem0, sem1):
    wid = lax.axis_index("s") * NCORES + lax.axis_index("c")
    base = wid * BPW
    sems = [sem0, sem1]

    pltpu.sync_copy(stm_hbm.at[pl.ds(base, BPW)], stm_v)
    pltpu.sync_copy(bw_hbm, bw_v)
    pltpu.sync_copy(bb_hbm, bb_v)
    pltpu.sync_copy(ow_hbm, ow_v)

    def run_phase(feat_hbm, w_hbm, b_v, phase_pd):
        pltpu.sync_copy(feat_hbm.at[pl.ds(base * ACTIVE, BPW * ACTIVE)],
                        idx_v)

        def issue(g, k):
            pltpu.async_copy(w_hbm.at[idx_v.at[pl.ds(g * GROWS, GROWS)]],
                             buf.at[k], sems[k])

        def wait(g, k):
            pltpu.make_async_copy(w_hbm.at[idx_v.at[pl.ds(g * GROWS, GROWS)]],
                                  buf.at[k], sems[k]).wait()

        for k in range(NBUF):
            issue(k, k)

        @pl.loop(0, 2 * BPW, step=2)
        def _row(g0):
            i = lax.div(g0, 2)
            for k in range(2):
                g = g0 + k
                wait(g, k)
                bufref = buf.at[k]

                if k == 0:
                    def chunk0(c, carry):
                        col = c * 16
                        rows = [bufref[r, pl.ds(col, 16)]
                                for r in range(GROWS)]
                        hrow_v[pl.ds(col, 16)] = _tree_sum(rows)
                        return carry

                    lax.fori_loop(0, NCHUNK, chunk0, 0)
                else:
                    def chunk1(c, carry):
                        r1, r2 = carry
                        col = c * 16
                        rows = [bufref[r, pl.ds(col, 16)]
                                for r in range(GROWS)]
                        h = (_tree_sum(rows) + hrow_v[pl.ds(col, 16)]
                             + b_v[pl.ds(col, 16)])
                        f = jnp.clip(h, 0.0, 1.0)
                        f = f * f
                        w1c = ow_v[pl.ds(col, 16)]
                        w2c = ow_v[pl.ds(HIDDEN + col, 16)]
                        return r1 + f * w1c, r2 + f * w2c

                    zero = jnp.zeros((16,), jnp.float32)
                    r1, r2 = lax.fori_loop(0, NCHUNK, chunk1, (zero, zero))
                    pd1, pd2 = phase_pd
                    pd1[i, :] = r1
                    pd2[i, :] = r2

                nxt = g + 2

                @pl.when(nxt < 2 * BPW)
                def _():
                    issue(nxt, k)

    run_phase(fw_hbm, ww_hbm, bw_v, (pw1_v, pw2_v))
    run_phase(fb_hbm, wb_hbm, bb_v, (pb1_v, pb2_v))

    lane = lax.iota(jnp.int32, 16)

    @pl.loop(0, BPW, step=16)
    def _group(off):
        wf = jnp.zeros((16,), jnp.float32)
        bf = jnp.zeros((16,), jnp.float32)
        for r in range(16):
            i = off + r
            s1 = _sum_lanes(pw1_v[i, :] + pb2_v[i, :])
            s2 = _sum_lanes(pb1_v[i, :] + pw2_v[i, :])
            wf = jnp.where(lane == r, s1, wf)
            bf = jnp.where(lane == r, s2, bf)
        sl = pl.ds(off, 16)
        s = stm_v[sl].astype(jnp.float32)
        out_v[sl] = s * wf + (1.0 - s) * bf

    pltpu.sync_copy(out_v, out_hbm.at[pl.ds(base, BPW)])


@jax.jit
def _run(fw_flat, fb_flat, stm_i, ww, bw, wb, bb, ow_flat):
    kfun = pl.kernel(
        _sc_body,
        out_type=jax.ShapeDtypeStruct((BATCH,), jnp.float32),
        mesh=plsc.VectorSubcoreMesh(core_axis_name="c", subcore_axis_name="s"),
        scratch_types=[
            pltpu.VMEM((BPW * ACTIVE,), jnp.int32),
            pltpu.VMEM((BPW,), jnp.int32),
            pltpu.VMEM((HIDDEN,), jnp.float32),
            pltpu.VMEM((HIDDEN,), jnp.float32),
            pltpu.VMEM((2 * HIDDEN,), jnp.float32),
            pltpu.VMEM((BPW, 16), jnp.float32),
            pltpu.VMEM((BPW, 16), jnp.float32),
            pltpu.VMEM((BPW, 16), jnp.float32),
            pltpu.VMEM((BPW, 16), jnp.float32),
            pltpu.VMEM((BPW,), jnp.float32),
            pltpu.VMEM((HIDDEN,), jnp.float32),
            pltpu.VMEM((NBUF, GROWS, HIDDEN), jnp.float32),
            pltpu.SemaphoreType.DMA,
            pltpu.SemaphoreType.DMA,
        ],
    )
    return kfun(fw_flat, fb_flat, stm_i, ww, bw, wb, bb, ow_flat)


def kernel(features_tensor_white, features_tensor_black, is_white_stm_tensor,
           ft_white_W, ft_white_b, ft_black_W, ft_black_b, out_W, out_b):
    stm_i = is_white_stm_tensor.astype(jnp.int32).reshape(BATCH)
    ow_flat = out_W.reshape(2 * HIDDEN)
    fw_flat = features_tensor_white.reshape(BATCH * ACTIVE)
    fb_flat = features_tensor_black.reshape(BATCH * ACTIVE)
    raw = _run(fw_flat, fb_flat, stm_i,
               ft_white_W, ft_white_b, ft_black_W, ft_black_b, ow_flat)
    return (raw + out_b).reshape(BATCH, 1)
